# Initial kernel scaffold; baseline (speedup 1.0000x reference)
import jax
import jax.numpy as jnp
from jax.experimental import pallas as pl


def kernel(x, pos, edge_index, edge_attr, batch, mu_r_norm, edge_w1, edge_b1, edge_w2, edge_b2, coor_w1, coor_b1, coor_w2, coor_b2, node_w1, node_b1, node_w2, node_b2, lin_w, lin_b):
    raise NotImplementedError("write your pallas kernel here")


# speedup vs baseline: 2.1283x; 2.1283x over previous
import functools

import jax
import jax.numpy as jnp
from jax import lax
from jax.experimental import pallas as pl
from jax.experimental.pallas import tpu as pltpu
from jax.experimental.pallas import tpu_sc as plsc

F32 = jnp.float32

_NC = 2
_NS = 16
_NW = _NC * _NS
_C = 128
_BN = 1024
_BE = 512

_SC_UNTILED = pltpu.CompilerParams(use_tc_tiling_on_sc=False)


def _silu(v):
    return v * jax.nn.sigmoid(v)


def _make_gather(e_pad, k_chunks, width, params):
    w_per = k_chunks * _C
    mesh = plsc.VectorSubcoreMesh(core_axis_name="c", subcore_axis_name="s")

    @functools.partial(
        pl.kernel,
        mesh=mesh,
        out_type=(
            jax.ShapeDtypeStruct((e_pad, width), F32),
            jax.ShapeDtypeStruct((e_pad, width), F32),
        ),
        scratch_types=[
            pltpu.VMEM((_C,), jnp.int32),
            pltpu.VMEM((_C, width), F32),
            pltpu.VMEM((_C,), jnp.int32),
            pltpu.VMEM((_C, width), F32),
            pltpu.SemaphoreType.DMA,
            pltpu.SemaphoreType.DMA,
        ],
        compiler_params=params,
    )
    def gather_k(ta_hbm, tb_hbm, src_hbm, dst_hbm, ga_hbm, gb_hbm,
                 idx1, buf1, idx2, buf2, sem1, sem2):
        wid = lax.axis_index("s") * _NC + lax.axis_index("c")
        base = wid * w_per

        def body(j, carry):
            off = base + j * _C
            pltpu.sync_copy(src_hbm.at[pl.ds(off, _C)], idx1)
            cp1 = pltpu.async_copy(ta_hbm.at[idx1], buf1, sem1)
            pltpu.sync_copy(dst_hbm.at[pl.ds(off, _C)], idx2)
            cp2 = pltpu.async_copy(tb_hbm.at[idx2], buf2, sem2)
            cp1.wait()
            pltpu.sync_copy(buf1, ga_hbm.at[pl.ds(off, _C)])
            cp2.wait()
            pltpu.sync_copy(buf2, gb_hbm.at[pl.ds(off, _C)])
            return carry

        lax.fori_loop(0, k_chunks, body, 0)

    return gather_k


def _edge_body(gs_ref, gd_ref, cs_ref, cd_ref, ea32_ref, w1p_ref,
               b1_ref, w2_ref, b2_ref, cw1_ref, cb1_ref, cw2p_ref, cb2_ref,
               m2_ref, aux_ref):
    relp = cs_ref[...] - cd_ref[...]
    sq = relp * relp
    rel_d = (sq[:, 0:1] + sq[:, 1:2]) + sq[:, 2:3]
    ea32 = ea32_ref[...]
    col32 = lax.broadcasted_iota(jnp.int32, ea32.shape, 1)
    x32 = jnp.where(col32 == 0, rel_d, ea32)
    e_in = jnp.concatenate([gs_ref[...], gd_ref[...], x32], axis=1)
    g = jnp.dot(e_in, w1p_ref[...], preferred_element_type=F32) + b1_ref[...]
    m1 = _silu(g)
    t = jnp.dot(m1, w2_ref[...], preferred_element_type=F32) + b2_ref[...]
    m2 = _silu(t)
    u = jnp.dot(m2, cw1_ref[...], preferred_element_type=F32) + cb1_ref[...]
    ch = _silu(u)
    cwc = jnp.dot(ch, cw2p_ref[...], preferred_element_type=F32)
    cw = cwc[:, 0:1] + cb2_ref[...]
    m2_ref[...] = m2
    col = lax.broadcasted_iota(jnp.int32, relp.shape, 1)
    aux_ref[...] = relp * cw + (col == 3).astype(F32)


def _edge_mlp(gs, gd, cs, cd, ea32P, w1p, b1, w2, b2, cw1, cb1, cw2p, cb2):
    e_pad = gs.shape[0]
    grid = (e_pad // _BE,)
    full = lambda r, c: pl.BlockSpec((r, c), lambda i: (0, 0))
    return pl.pallas_call(
        _edge_body,
        grid=grid,
        in_specs=[
            pl.BlockSpec((_BE, 128), lambda i: (i, 0)),
            pl.BlockSpec((_BE, 128), lambda i: (i, 0)),
            pl.BlockSpec((_BE, 16), lambda i: (i, 0)),
            pl.BlockSpec((_BE, 16), lambda i: (i, 0)),
            pl.BlockSpec((_BE, 32), lambda i: (i, 0)),
            full(288, 128), full(1, 128),
            full(128, 128), full(1, 128),
            full(128, 128), full(1, 128),
            full(128, 8), full(1, 1),
        ],
        out_specs=[
            pl.BlockSpec((_BE, 128), lambda i: (i, 0)),
            pl.BlockSpec((_BE, 16), lambda i: (i, 0)),
        ],
        out_shape=[
            jax.ShapeDtypeStruct((e_pad, 128), F32),
            jax.ShapeDtypeStruct((e_pad, 16), F32),
        ],
    )(gs, gd, cs, cd, ea32P, w1p, b1, w2, b2, cw1, cb1, cw2p, cb2)


def _make_scatter(npad, e_pad, k_chunks, width, params):
    w_per = k_chunks * _C
    stripe = npad // _NS
    mesh = plsc.VectorSubcoreMesh(core_axis_name="c", subcore_axis_name="s")

    @functools.partial(
        pl.kernel,
        mesh=mesh,
        out_type=jax.ShapeDtypeStruct((_NC, npad, width), F32),
        scratch_types=[
            pltpu.VMEM((_C,), jnp.int32),
            pltpu.VMEM((_C, width), F32),
            pltpu.VMEM_SHARED((npad, width), F32),
        ],
        compiler_params=params,
    )
    def scatter_k(rows_hbm, src_hbm, z_hbm, part_hbm, idxv, buf, acc):
        c = lax.axis_index("c")
        s = lax.axis_index("s")
        wid = s * _NC + c
        base = wid * w_per
        pltpu.sync_copy(z_hbm, acc.at[pl.ds(s * stripe, stripe)])
        plsc.subcore_barrier()

        def body(j, carry):
            off = base + j * _C
            pltpu.sync_copy(src_hbm.at[pl.ds(off, _C)], idxv)
            pltpu.sync_copy(rows_hbm.at[pl.ds(off, _C)], buf)
            pltpu.sync_copy(buf, acc.at[idxv], add=True)
            return carry

        lax.fori_loop(0, k_chunks, body, 0)
        plsc.subcore_barrier()
        pltpu.sync_copy(acc.at[pl.ds(s * stripe, stripe)],
                        part_hbm.at[c, pl.ds(s * stripe, stripe)])

    return scatter_k


def _node_body(pm0_ref, pm1_ref, pa0_ref, pa1_ref, feats_ref, coorsp_ref,
               nw1_ref, nb1_ref, nw2_ref, nb2_ref,
               feats_out_ref, coorsp_out_ref):
    m_agg = pm0_ref[0] + pm1_ref[0]
    aux = pa0_ref[0] + pa1_ref[0]
    col = lax.broadcasted_iota(jnp.int32, aux.shape, 1)
    cnt = jnp.sum(jnp.where(col == 3, aux, 0.0), axis=1, keepdims=True)
    csum = jnp.where(col < 3, aux, 0.0)
    cp = coorsp_ref[...]
    coorsp_out_ref[...] = cp + (cp + csum / jnp.maximum(cnt, 1.0))
    f = feats_ref[...]
    nx = jnp.concatenate([f, m_agg], axis=1)
    t = jnp.dot(nx, nw1_ref[...], preferred_element_type=F32) + nb1_ref[...]
    t = _silu(t)
    fo = jnp.dot(t, nw2_ref[...], preferred_element_type=F32) + nb2_ref[...]
    feats_out_ref[...] = f + fo


def _node_update(pm, pa, featsP, coorspP, nw1, nb1, nw2, nb2):
    npad = featsP.shape[0]
    grid = (npad // _BN,)
    full = lambda r, c: pl.BlockSpec((r, c), lambda i: (0, 0))
    return pl.pallas_call(
        _node_body,
        grid=grid,
        in_specs=[
            pl.BlockSpec((1, _BN, 128), lambda i: (0, i, 0)),
            pl.BlockSpec((1, _BN, 128), lambda i: (1, i, 0)),
            pl.BlockSpec((1, _BN, 16), lambda i: (0, i, 0)),
            pl.BlockSpec((1, _BN, 16), lambda i: (1, i, 0)),
            pl.BlockSpec((_BN, 128), lambda i: (i, 0)),
            pl.BlockSpec((_BN, 16), lambda i: (i, 0)),
            full(256, 256), full(1, 256),
            full(256, 128), full(1, 128),
        ],
        out_specs=[
            pl.BlockSpec((_BN, 128), lambda i: (i, 0)),
            pl.BlockSpec((_BN, 16), lambda i: (i, 0)),
        ],
        out_shape=[
            jax.ShapeDtypeStruct((npad, 128), F32),
            jax.ShapeDtypeStruct((npad, 16), F32),
        ],
    )(pm, pm, pa, pa, featsP, coorspP, nw1, nb1, nw2, nb2)


def _final_body(feats_ref, lw_ref, lb_ref, out_ref):
    out_ref[...] = (jnp.dot(feats_ref[...], lw_ref[...],
                            preferred_element_type=F32) + lb_ref[...])


def _final(featsP, lin_w, lin_b):
    npad = featsP.shape[0]
    grid = (npad // _BN,)
    return pl.pallas_call(
        _final_body,
        grid=grid,
        in_specs=[
            pl.BlockSpec((_BN, 128), lambda i: (i, 0)),
            pl.BlockSpec((128, 128), lambda i: (0, 0)),
            pl.BlockSpec((1, 128), lambda i: (0, 0)),
        ],
        out_specs=pl.BlockSpec((_BN, 128), lambda i: (i, 0)),
        out_shape=jax.ShapeDtypeStruct((npad, 128), F32),
    )(featsP, lin_w, lin_b)


def kernel(x, pos, edge_index, edge_attr, batch, mu_r_norm,
           edge_w1, edge_b1, edge_w2, edge_b2,
           coor_w1, coor_b1, coor_w2, coor_b2,
           node_w1, node_b1, node_w2, node_b2, lin_w, lin_b):
    n = x.shape[0]
    e = edge_index.shape[1]
    num_layers = edge_w1.shape[0]

    npad = ((n + 1 + _BN - 1) // _BN) * _BN
    k_chunks = -(-e // (_NW * _C))
    e_pad = _NW * _C * k_chunks

    feats0 = jnp.concatenate([x, mu_r_norm], axis=1)
    featsP = jnp.pad(feats0, ((0, npad - n), (0, 0)))
    coorspP = jnp.pad(pos, ((0, npad - n), (0, 13)))
    src = jnp.pad(edge_index[0].astype(jnp.int32), (0, e_pad - e),
                  constant_values=n)
    dst = jnp.pad(edge_index[1].astype(jnp.int32), (0, e_pad - e),
                  constant_values=n)
    ea32P = jnp.pad(edge_attr, ((0, e_pad - e), (1, 15)))

    zm = jnp.zeros((npad // _NS, 128), F32)
    za = jnp.zeros((npad // _NS, 16), F32)

    gather128 = _make_gather(e_pad, k_chunks, 128, None)
    gather16 = _make_gather(e_pad, k_chunks, 16, _SC_UNTILED)
    scatter128 = _make_scatter(npad, e_pad, k_chunks, 128, None)
    scatter16 = _make_scatter(npad, e_pad, k_chunks, 16, _SC_UNTILED)

    for l in range(num_layers):
        w1p = jnp.pad(edge_w1[l], ((0, 15), (0, 0)))
        b1 = edge_b1[l][None]
        w2 = edge_w2[l]
        b2 = edge_b2[l][None]
        cw1 = coor_w1[l]
        cb1 = coor_b1[l][None]
        cw2p = jnp.pad(coor_w2[l], ((0, 0), (0, 7)))
        cb2 = coor_b2[l][None]
        nw1 = node_w1[l]
        nb1 = node_b1[l][None]
        nw2 = node_w2[l]
        nb2 = node_b2[l][None]

        gs, gd = gather128(featsP, featsP, src, dst)
        cs, cd = gather16(coorspP, coorspP, src, dst)
        m2, aux = _edge_mlp(gs, gd, cs, cd, ea32P, w1p, b1, w2, b2,
                            cw1, cb1, cw2p, cb2)
        pm = scatter128(m2, src, zm)
        pa = scatter16(aux, src, za)
        featsP, coorspP = _node_update(pm, pa, featsP, coorspP,
                                       nw1, nb1, nw2, nb2)

    out = _final(featsP, lin_w, lin_b[None])
    return out[:n]
